# Initial kernel scaffold; baseline (speedup 1.0000x reference)
import jax
import jax.numpy as jnp
from jax.experimental import pallas as pl


def kernel(x, edge_index, y, W1, b1, W2, b2):
    raise NotImplementedError("write your pallas kernel here")


# speedup vs baseline: 18.8530x; 18.8530x over previous
import functools

import jax
import jax.numpy as jnp
from jax import lax
from jax.experimental import pallas as pl
from jax.experimental.pallas import tpu as pltpu
from jax.experimental.pallas import tpu_sc as plsc

N = 50000
E = 1600000
F_IN = 1433
H = 128
C = 7

NT = 16
NC = 2
LW = 128
KC = 8
KC1 = 4
R = 12544
E_PAD = R * LW
R_TILE = R // NT
R_HALF_TILE = R // NC // NT
ACC_ROWS = 51200
ZROWS = ACC_ROWS // NT
N_PAD = 50048
OUT_TILE = N_PAD // NT

BM = 1000


def _mesh():
    return plsc.VectorSubcoreMesh(core_axis_name="c", subcore_axis_name="s")


def _deg_body(dstr_hbm, zeros_hbm, out_hbm, ones_v, dst_v, acc):
    cid = lax.axis_index("c")
    tid = lax.axis_index("s")
    o16 = jnp.ones((16,), jnp.float32)
    for q in range(LW // 16):
        ones_v[pl.ds(q * 16, 16)] = o16

    pltpu.sync_copy(zeros_hbm, acc.at[pl.ds(tid * ZROWS, ZROWS)])
    plsc.subcore_barrier()

    def chunk(ci, _):
        row0 = cid * (R // NC) + tid * R_HALF_TILE + ci * KC
        pltpu.sync_copy(dstr_hbm.at[pl.ds(row0, KC)], dst_v)
        for j in range(KC):
            pltpu.sync_copy(ones_v, acc.at[dst_v.at[j]], add=True)
        return 0
    lax.fori_loop(0, R_HALF_TILE // KC, chunk, 0, unroll=False)
    plsc.subcore_barrier()

    pltpu.sync_copy(
        acc.at[pl.ds(tid * ZROWS, ZROWS)],
        out_hbm.at[pl.ds(cid * ACC_ROWS + tid * ZROWS, ZROWS)],
    )


def _deg_call(dstr, zeros1):
    return pl.kernel(
        _deg_body,
        out_type=jax.ShapeDtypeStruct((NC * ACC_ROWS,), jnp.float32),
        mesh=_mesh(),
        scratch_types=[
            pltpu.VMEM((LW,), jnp.float32),
            pltpu.VMEM((KC, LW), jnp.int32),
            pltpu.VMEM_SHARED((ACC_ROWS,), jnp.float32),
        ],
        compiler_params=pltpu.CompilerParams(use_tc_tiling_on_sc=False),
        name="sc_deg_hist",
    )(dstr, zeros1)


def _agg1_body(h1s_hbm, srcr_hbm, dstr_hbm, zeros_hbm, out_hbm,
               src_v, dst_v, rows_v, acc, sem):
    cid = lax.axis_index("c")
    tid = lax.axis_index("s")

    for p in range(2):
        s_idx = cid * 2 + p
        off = s_idx * N

        pltpu.sync_copy(zeros_hbm, acc.at[pl.ds(tid * ZROWS, ZROWS)])
        plsc.subcore_barrier()

        def chunk(ci, _):
            row0 = tid * R_TILE + ci * KC1
            pltpu.sync_copy(srcr_hbm.at[pl.ds(row0, KC1)], src_v)
            pltpu.sync_copy(dstr_hbm.at[pl.ds(row0, KC1)], dst_v)
            for j in range(KC1):
                for q in range(LW // 16):
                    v = src_v[j, pl.ds(q * 16, 16)]
                    src_v[j, pl.ds(q * 16, 16)] = v + off
            cps = [
                pltpu.async_copy(h1s_hbm.at[src_v.at[j]], rows_v.at[j], sem)
                for j in range(KC1)
            ]
            for cp in cps:
                cp.wait()
            for j in range(KC1):
                pltpu.sync_copy(rows_v.at[j], acc.at[dst_v.at[j]], add=True)
            return 0
        lax.fori_loop(0, R_TILE // KC1, chunk, 0, unroll=False)
        plsc.subcore_barrier()

        pltpu.sync_copy(
            acc.at[pl.ds(tid * OUT_TILE, OUT_TILE)],
            out_hbm.at[pl.ds(s_idx * N_PAD + tid * OUT_TILE, OUT_TILE)],
        )
        plsc.subcore_barrier()


def _agg1_call(h1s2d, srcr, dstr, zeros32):
    return pl.kernel(
        _agg1_body,
        out_type=jax.ShapeDtypeStruct((4 * N_PAD, 32), jnp.float32),
        mesh=_mesh(),
        scratch_types=[
            pltpu.VMEM((KC1, LW), jnp.int32),
            pltpu.VMEM((KC1, LW), jnp.int32),
            pltpu.VMEM((KC1, LW, 32), jnp.float32),
            pltpu.VMEM_SHARED((ACC_ROWS, 32), jnp.float32),
            pltpu.SemaphoreType.DMA,
        ],
        compiler_params=pltpu.CompilerParams(use_tc_tiling_on_sc=False),
        name="sc_agg1",
    )(h1s2d, srcr, dstr, zeros32)


def _agg2_body(h2p_hbm, srcr_hbm, dstr_hbm, zeros_hbm, out_hbm,
               src_v, dst_v, rows_v, acc, sem):
    cid = lax.axis_index("c")
    tid = lax.axis_index("s")

    pltpu.sync_copy(zeros_hbm, acc.at[pl.ds(tid * ZROWS, ZROWS)])
    plsc.subcore_barrier()

    def chunk(ci, _):
        row0 = cid * (R // NC) + tid * R_HALF_TILE + ci * KC
        pltpu.sync_copy(srcr_hbm.at[pl.ds(row0, KC)], src_v)
        pltpu.sync_copy(dstr_hbm.at[pl.ds(row0, KC)], dst_v)
        cps = [
            pltpu.async_copy(h2p_hbm.at[src_v.at[j]], rows_v.at[j], sem)
            for j in range(KC)
        ]
        for cp in cps:
            cp.wait()
        for j in range(KC):
            pltpu.sync_copy(rows_v.at[j], acc.at[dst_v.at[j]], add=True)
        return 0
    lax.fori_loop(0, R_HALF_TILE // KC, chunk, 0, unroll=False)
    plsc.subcore_barrier()

    pltpu.sync_copy(
        acc.at[pl.ds(tid * OUT_TILE, OUT_TILE)],
        out_hbm.at[pl.ds(cid * N_PAD + tid * OUT_TILE, OUT_TILE)],
    )


def _agg2_call(h2p, srcr, dstr, zeros8):
    return pl.kernel(
        _agg2_body,
        out_type=jax.ShapeDtypeStruct((NC * N_PAD, 8), jnp.float32),
        mesh=_mesh(),
        scratch_types=[
            pltpu.VMEM((KC, LW), jnp.int32),
            pltpu.VMEM((KC, LW), jnp.int32),
            pltpu.VMEM((KC, LW, 8), jnp.float32),
            pltpu.VMEM_SHARED((ACC_ROWS, 8), jnp.float32),
            pltpu.SemaphoreType.DMA,
        ],
        compiler_params=pltpu.CompilerParams(use_tc_tiling_on_sc=False),
        name="sc_agg2",
    )(h2p, srcr, dstr, zeros8)


def _mm1_body(x_ref, w_ref, o_ref):
    o_ref[...] = jnp.dot(x_ref[...], w_ref[...],
                         preferred_element_type=jnp.float32)


def _mm1_call(x, W1):
    return pl.pallas_call(
        _mm1_body,
        grid=(N // BM,),
        in_specs=[
            pl.BlockSpec((BM, F_IN), lambda i: (i, 0)),
            pl.BlockSpec((F_IN, H), lambda i: (0, 0)),
        ],
        out_specs=pl.BlockSpec((BM, H), lambda i: (i, 0)),
        out_shape=jax.ShapeDtypeStruct((N, H), jnp.float32),
        name="tc_mm1",
    )(x, W1)


def _scale_body(h1_ref, deg_ref, h1s_ref, dinv_ref):
    deg = deg_ref[:, 0:1] + deg_ref[:, 1:2] + 1.0
    dv = lax.rsqrt(deg)
    dinv_ref[...] = dv
    hp = h1_ref[...] * dv
    for s in range(4):
        h1s_ref[s] = hp[:, 32 * s:32 * s + 32]


def _scale_call(h1, deg2t):
    return pl.pallas_call(
        _scale_body,
        grid=(N // BM,),
        in_specs=[
            pl.BlockSpec((BM, H), lambda i: (i, 0)),
            pl.BlockSpec((BM, 2), lambda i: (i, 0)),
        ],
        out_specs=[
            pl.BlockSpec((4, BM, 32), lambda i: (0, i, 0)),
            pl.BlockSpec((BM, 1), lambda i: (i, 0)),
        ],
        out_shape=[
            jax.ShapeDtypeStruct((4, N, 32), jnp.float32),
            jax.ShapeDtypeStruct((N, 1), jnp.float32),
        ],
        name="tc_scale_slice",
    )(h1, deg2t)


def _layer2_body(agg_ref, h1s_ref, dinv_ref, b1_ref, w2_ref, o_ref):
    dv = dinv_ref[...]
    a = jnp.concatenate(
        [agg_ref[s] + h1s_ref[s] for s in range(4)], axis=1)
    z1 = jnp.maximum(a * dv + b1_ref[...][None, :], 0.0)
    h2 = jnp.dot(z1, w2_ref[...], preferred_element_type=jnp.float32)
    o_ref[...] = h2 * dv


def _layer2_call(agg1, h1s, dinv, b1, W2p):
    return pl.pallas_call(
        _layer2_body,
        grid=(N // BM,),
        in_specs=[
            pl.BlockSpec((4, BM, 32), lambda i: (0, i, 0)),
            pl.BlockSpec((4, BM, 32), lambda i: (0, i, 0)),
            pl.BlockSpec((BM, 1), lambda i: (i, 0)),
            pl.BlockSpec((H,), lambda i: (0,)),
            pl.BlockSpec((H, 8), lambda i: (0, 0)),
        ],
        out_specs=pl.BlockSpec((BM, 8), lambda i: (i, 0)),
        out_shape=jax.ShapeDtypeStruct((N, 8), jnp.float32),
        name="tc_layer2",
    )(agg1, h1s, dinv, b1, W2p)


def _final_body(agg2_ref, h2p_ref, dinv_ref, b2_ref, o_ref):
    s = agg2_ref[0] + agg2_ref[1] + h2p_ref[...]
    o_ref[...] = s * dinv_ref[...] + b2_ref[...][None, :]


def _final_call(agg2, h2p, dinv, b2p):
    return pl.pallas_call(
        _final_body,
        grid=(N // BM,),
        in_specs=[
            pl.BlockSpec((2, BM, 8), lambda i: (0, i, 0)),
            pl.BlockSpec((BM, 8), lambda i: (i, 0)),
            pl.BlockSpec((BM, 1), lambda i: (i, 0)),
            pl.BlockSpec((8,), lambda i: (0,)),
        ],
        out_specs=pl.BlockSpec((BM, 8), lambda i: (i, 0)),
        out_shape=jax.ShapeDtypeStruct((N, 8), jnp.float32),
        name="tc_final",
    )(agg2, h2p, dinv, b2p)


def kernel(x, edge_index, y, W1, b1, W2, b2):
    ei = edge_index.astype(jnp.int32)
    pad = E_PAD - E
    srcr = jnp.concatenate(
        [ei[0], jnp.zeros((pad,), jnp.int32)]).reshape(R, LW)
    dstr = jnp.concatenate(
        [ei[1], jnp.full((pad,), N, jnp.int32)]).reshape(R, LW)
    W2p = jnp.pad(W2, ((0, 0), (0, 1)))
    b2p = jnp.pad(b2, (0, 1))
    zeros1 = jnp.zeros((ZROWS,), jnp.float32)
    zeros32 = jnp.zeros((ZROWS, 32), jnp.float32)
    zeros8 = jnp.zeros((ZROWS, 8), jnp.float32)

    h1 = _mm1_call(x, W1)
    degflat = _deg_call(dstr, zeros1)
    deg2t = degflat.reshape(NC, ACC_ROWS)[:, :N].T
    h1s, dinv = _scale_call(h1, deg2t)
    agg1 = _agg1_call(h1s.reshape(4 * N, 32), srcr, dstr,
                      zeros32).reshape(4, N_PAD, 32)[:, :N, :]
    h2p = _layer2_call(agg1, h1s, dinv, b1, W2p)
    agg2 = _agg2_call(h2p, srcr, dstr, zeros8).reshape(NC, N_PAD, 8)[:, :N, :]
    out8 = _final_call(agg2, h2p, dinv, b2p)
    return out8[:, :C]


# speedup vs baseline: 19.4259x; 1.0304x over previous
import functools

import jax
import jax.numpy as jnp
from jax import lax
from jax.experimental import pallas as pl
from jax.experimental.pallas import tpu as pltpu
from jax.experimental.pallas import tpu_sc as plsc

N = 50000
E = 1600000
F_IN = 1433
H = 128
C = 7

NT = 16
NC = 2
LW = 128
KC = 8
KC1 = 4
R = 12544
E_PAD = R * LW
R_TILE = R // NT
R_HALF_TILE = R // NC // NT
ACC_ROWS = 51200
ZROWS = ACC_ROWS // NT
N_PAD = 50048
OUT_TILE = N_PAD // NT

BM = 1000


def _mesh():
    return plsc.VectorSubcoreMesh(core_axis_name="c", subcore_axis_name="s")


def _deg_body(dstr_hbm, zeros_hbm, out_hbm, ones_v, dst_v, acc, sem):
    cid = lax.axis_index("c")
    tid = lax.axis_index("s")
    o16 = jnp.ones((16,), jnp.float32)
    for q in range(LW // 16):
        ones_v[pl.ds(q * 16, 16)] = o16

    pltpu.sync_copy(zeros_hbm, acc.at[pl.ds(tid * ZROWS, ZROWS)])
    plsc.subcore_barrier()

    def chunk(ci, _):
        row0 = cid * (R // NC) + tid * R_HALF_TILE + ci * KC
        pltpu.sync_copy(dstr_hbm.at[pl.ds(row0, KC)], dst_v)
        cps = [
            pltpu.async_copy(ones_v, acc.at[dst_v.at[j]], sem, add=True)
            for j in range(KC)
        ]
        for cp in cps:
            cp.wait()
        return 0
    lax.fori_loop(0, R_HALF_TILE // KC, chunk, 0, unroll=False)
    plsc.subcore_barrier()

    pltpu.sync_copy(
        acc.at[pl.ds(tid * ZROWS, ZROWS)],
        out_hbm.at[pl.ds(cid * ACC_ROWS + tid * ZROWS, ZROWS)],
    )


def _deg_call(dstr, zeros1):
    return pl.kernel(
        _deg_body,
        out_type=jax.ShapeDtypeStruct((NC * ACC_ROWS,), jnp.float32),
        mesh=_mesh(),
        scratch_types=[
            pltpu.VMEM((LW,), jnp.float32),
            pltpu.VMEM((KC, LW), jnp.int32),
            pltpu.VMEM_SHARED((ACC_ROWS,), jnp.float32),
            pltpu.SemaphoreType.DMA,
        ],
        compiler_params=pltpu.CompilerParams(use_tc_tiling_on_sc=False),
        name="sc_deg_hist",
    )(dstr, zeros1)


def _agg1_body(h1s_hbm, srcr4_hbm, dstr_hbm, zeros_hbm, out_hbm,
               src_v, dst_v, rows_v, acc, sem, sem2):
    cid = lax.axis_index("c")
    tid = lax.axis_index("s")

    for p in range(2):
        s_idx = cid * 2 + p

        pltpu.sync_copy(zeros_hbm, acc.at[pl.ds(tid * ZROWS, ZROWS)])
        plsc.subcore_barrier()

        def chunk(ci, _):
            row0 = tid * R_TILE + ci * KC1
            pltpu.sync_copy(srcr4_hbm.at[pl.ds(s_idx * R + row0, KC1)], src_v)
            pltpu.sync_copy(dstr_hbm.at[pl.ds(row0, KC1)], dst_v)
            cps = [
                pltpu.async_copy(h1s_hbm.at[src_v.at[j]], rows_v.at[j], sem)
                for j in range(KC1)
            ]
            for cp in cps:
                cp.wait()
            sps = [
                pltpu.async_copy(rows_v.at[j], acc.at[dst_v.at[j]], sem2,
                                 add=True)
                for j in range(KC1)
            ]
            for sp in sps:
                sp.wait()
            return 0
        lax.fori_loop(0, R_TILE // KC1, chunk, 0, unroll=False)
        plsc.subcore_barrier()

        pltpu.sync_copy(
            acc.at[pl.ds(tid * OUT_TILE, OUT_TILE)],
            out_hbm.at[pl.ds(s_idx * N_PAD + tid * OUT_TILE, OUT_TILE)],
        )
        plsc.subcore_barrier()


def _agg1_call(h1s2d, srcr4, dstr, zeros32):
    return pl.kernel(
        _agg1_body,
        out_type=jax.ShapeDtypeStruct((4 * N_PAD, 32), jnp.float32),
        mesh=_mesh(),
        scratch_types=[
            pltpu.VMEM((KC1, LW), jnp.int32),
            pltpu.VMEM((KC1, LW), jnp.int32),
            pltpu.VMEM((KC1, LW, 32), jnp.float32),
            pltpu.VMEM_SHARED((ACC_ROWS, 32), jnp.float32),
            pltpu.SemaphoreType.DMA,
            pltpu.SemaphoreType.DMA,
        ],
        compiler_params=pltpu.CompilerParams(use_tc_tiling_on_sc=False),
        name="sc_agg1",
    )(h1s2d, srcr4, dstr, zeros32)


def _agg2_body(h2p_hbm, srcr_hbm, dstr_hbm, zeros_hbm, out_hbm,
               src_v, dst_v, rows_v, acc, sem, sem2):
    cid = lax.axis_index("c")
    tid = lax.axis_index("s")

    pltpu.sync_copy(zeros_hbm, acc.at[pl.ds(tid * ZROWS, ZROWS)])
    plsc.subcore_barrier()

    def chunk(ci, _):
        row0 = cid * (R // NC) + tid * R_HALF_TILE + ci * KC
        pltpu.sync_copy(srcr_hbm.at[pl.ds(row0, KC)], src_v)
        pltpu.sync_copy(dstr_hbm.at[pl.ds(row0, KC)], dst_v)
        cps = [
            pltpu.async_copy(h2p_hbm.at[src_v.at[j]], rows_v.at[j], sem)
            for j in range(KC)
        ]
        for cp in cps:
            cp.wait()
        sps = [
            pltpu.async_copy(rows_v.at[j], acc.at[dst_v.at[j]], sem2, add=True)
            for j in range(KC)
        ]
        for sp in sps:
            sp.wait()
        return 0
    lax.fori_loop(0, R_HALF_TILE // KC, chunk, 0, unroll=False)
    plsc.subcore_barrier()

    pltpu.sync_copy(
        acc.at[pl.ds(tid * OUT_TILE, OUT_TILE)],
        out_hbm.at[pl.ds(cid * N_PAD + tid * OUT_TILE, OUT_TILE)],
    )


def _agg2_call(h2p, srcr, dstr, zeros8):
    return pl.kernel(
        _agg2_body,
        out_type=jax.ShapeDtypeStruct((NC * N_PAD, 8), jnp.float32),
        mesh=_mesh(),
        scratch_types=[
            pltpu.VMEM((KC, LW), jnp.int32),
            pltpu.VMEM((KC, LW), jnp.int32),
            pltpu.VMEM((KC, LW, 8), jnp.float32),
            pltpu.VMEM_SHARED((ACC_ROWS, 8), jnp.float32),
            pltpu.SemaphoreType.DMA,
            pltpu.SemaphoreType.DMA,
        ],
        compiler_params=pltpu.CompilerParams(use_tc_tiling_on_sc=False),
        name="sc_agg2",
    )(h2p, srcr, dstr, zeros8)


def _mm1_body(x_ref, w_ref, o_ref):
    o_ref[...] = jnp.dot(x_ref[...], w_ref[...],
                         preferred_element_type=jnp.float32)


def _mm1_call(x, W1):
    return pl.pallas_call(
        _mm1_body,
        grid=(N // BM,),
        in_specs=[
            pl.BlockSpec((BM, F_IN), lambda i: (i, 0)),
            pl.BlockSpec((F_IN, H), lambda i: (0, 0)),
        ],
        out_specs=pl.BlockSpec((BM, H), lambda i: (i, 0)),
        out_shape=jax.ShapeDtypeStruct((N, H), jnp.float32),
        name="tc_mm1",
    )(x, W1)


def _scale_body(h1_ref, deg_ref, h1s_ref, dinv_ref):
    deg = deg_ref[:, 0:1] + deg_ref[:, 1:2] + 1.0
    dv = lax.rsqrt(deg)
    dinv_ref[...] = dv
    hp = h1_ref[...] * dv
    for s in range(4):
        h1s_ref[s] = hp[:, 32 * s:32 * s + 32]


def _scale_call(h1, deg2t):
    return pl.pallas_call(
        _scale_body,
        grid=(N // BM,),
        in_specs=[
            pl.BlockSpec((BM, H), lambda i: (i, 0)),
            pl.BlockSpec((BM, 2), lambda i: (i, 0)),
        ],
        out_specs=[
            pl.BlockSpec((4, BM, 32), lambda i: (0, i, 0)),
            pl.BlockSpec((BM, 1), lambda i: (i, 0)),
        ],
        out_shape=[
            jax.ShapeDtypeStruct((4, N, 32), jnp.float32),
            jax.ShapeDtypeStruct((N, 1), jnp.float32),
        ],
        name="tc_scale_slice",
    )(h1, deg2t)


def _layer2_body(agg_ref, h1s_ref, dinv_ref, b1_ref, w2_ref, o_ref):
    dv = dinv_ref[...]
    a = jnp.concatenate(
        [agg_ref[s] + h1s_ref[s] for s in range(4)], axis=1)
    z1 = jnp.maximum(a * dv + b1_ref[...][None, :], 0.0)
    h2 = jnp.dot(z1, w2_ref[...], preferred_element_type=jnp.float32)
    o_ref[...] = h2 * dv


def _layer2_call(agg1, h1s, dinv, b1, W2p):
    return pl.pallas_call(
        _layer2_body,
        grid=(N // BM,),
        in_specs=[
            pl.BlockSpec((4, BM, 32), lambda i: (0, i, 0)),
            pl.BlockSpec((4, BM, 32), lambda i: (0, i, 0)),
            pl.BlockSpec((BM, 1), lambda i: (i, 0)),
            pl.BlockSpec((H,), lambda i: (0,)),
            pl.BlockSpec((H, 8), lambda i: (0, 0)),
        ],
        out_specs=pl.BlockSpec((BM, 8), lambda i: (i, 0)),
        out_shape=jax.ShapeDtypeStruct((N, 8), jnp.float32),
        name="tc_layer2",
    )(agg1, h1s, dinv, b1, W2p)


def _final_body(agg2_ref, h2p_ref, dinv_ref, b2_ref, o_ref):
    s = agg2_ref[0] + agg2_ref[1] + h2p_ref[...]
    o_ref[...] = s * dinv_ref[...] + b2_ref[...][None, :]


def _final_call(agg2, h2p, dinv, b2p):
    return pl.pallas_call(
        _final_body,
        grid=(N // BM,),
        in_specs=[
            pl.BlockSpec((2, BM, 8), lambda i: (0, i, 0)),
            pl.BlockSpec((BM, 8), lambda i: (i, 0)),
            pl.BlockSpec((BM, 1), lambda i: (i, 0)),
            pl.BlockSpec((8,), lambda i: (0,)),
        ],
        out_specs=pl.BlockSpec((BM, 8), lambda i: (i, 0)),
        out_shape=jax.ShapeDtypeStruct((N, 8), jnp.float32),
        name="tc_final",
    )(agg2, h2p, dinv, b2p)


def kernel(x, edge_index, y, W1, b1, W2, b2):
    ei = edge_index.astype(jnp.int32)
    pad = E_PAD - E
    srcr = jnp.concatenate(
        [ei[0], jnp.zeros((pad,), jnp.int32)]).reshape(R, LW)
    dstr = jnp.concatenate(
        [ei[1], jnp.full((pad,), N, jnp.int32)]).reshape(R, LW)
    W2p = jnp.pad(W2, ((0, 0), (0, 1)))
    b2p = jnp.pad(b2, (0, 1))
    zeros1 = jnp.zeros((ZROWS,), jnp.float32)
    zeros32 = jnp.zeros((ZROWS, 32), jnp.float32)
    zeros8 = jnp.zeros((ZROWS, 8), jnp.float32)

    h1 = _mm1_call(x, W1)
    degflat = _deg_call(dstr, zeros1)
    deg2t = degflat.reshape(NC, ACC_ROWS)[:, :N].T
    h1s, dinv = _scale_call(h1, deg2t)
    srcr4 = (srcr[None] + (jnp.arange(4, dtype=jnp.int32) * N)[:, None, None]
             ).reshape(4 * R, LW)
    agg1 = _agg1_call(h1s.reshape(4 * N, 32), srcr4, dstr,
                      zeros32).reshape(4, N_PAD, 32)[:, :N, :]
    h2p = _layer2_call(agg1, h1s, dinv, b1, W2p)
    agg2 = _agg2_call(h2p, srcr, dstr, zeros8).reshape(NC, N_PAD, 8)[:, :N, :]
    out8 = _final_call(agg2, h2p, dinv, b2p)
    return out8[:, :C]
